# Initial kernel scaffold; baseline (speedup 1.0000x reference)
import jax
import jax.numpy as jnp
from jax.experimental import pallas as pl


def kernel(x, edge_index, W_ih, W_hh, b_ih, b_hh, gc1_W, gc1_b, gc2_W, gc2_b, fc_W, fc_b):
    raise NotImplementedError("write your pallas kernel here")


# speedup vs baseline: 11.2555x; 11.2555x over previous
import functools

import jax
import jax.numpy as jnp
from jax import lax
from jax.experimental import pallas as pl
from jax.experimental.pallas import tpu as pltpu
from jax.experimental.pallas import tpu_sc as plsc

N = 50000
T = 12
H = 64
HOR = 12
E = 800000

NTILE = 16
NCORE = 2
WB = 3128
NP = NTILE * WB
CH = 128

EP_AGG = E // NTILE
NF_AGG = EP_AGG // CH
REM_AGG = EP_AGG - NF_AGG * CH

EP_DEG = E // (NTILE * NCORE)
NF_DEG = EP_DEG // CH
REM_DEG = EP_DEG - NF_DEG * CH

BN = 1000
GRID = N // BN

_mesh = plsc.VectorSubcoreMesh(core_axis_name="c", subcore_axis_name="s")


def _deg_body(dst_r, out0_r, out1_r, ones_b, idx_b, idx_rb, sem, deg_sh):
    c = lax.axis_index("c")
    s = lax.axis_index("s")
    z16 = jnp.zeros((16,), jnp.float32)
    o16 = jnp.ones((16,), jnp.float32)

    def fill(i, carry):
        ones_b[i, pl.ds(0, 16)] = z16
        return carry

    lax.fori_loop(0, CH, fill, 0)

    row0 = s * WB

    def zcopy(k, carry):
        pltpu.sync_copy(ones_b, deg_sh.at[pl.ds(row0 + k * CH, CH)])
        return carry

    lax.fori_loop(0, WB // CH, zcopy, 0)
    pltpu.sync_copy(ones_b.at[pl.ds(0, WB - (WB // CH) * CH)],
                    deg_sh.at[pl.ds(row0 + (WB // CH) * CH, WB - (WB // CH) * CH)])

    def refill(i, carry):
        ones_b[i, pl.ds(0, 16)] = o16
        return carry

    lax.fori_loop(0, CH, refill, 0)
    plsc.subcore_barrier()

    ebase = (c * NTILE + s) * EP_DEG

    def step(j, carry):
        pltpu.sync_copy(dst_r.at[pl.ds(ebase + j * CH, CH)], idx_b)
        pltpu.sync_copy(ones_b, deg_sh.at[idx_b], add=True)
        return carry

    lax.fori_loop(0, NF_DEG, step, 0)
    pltpu.sync_copy(dst_r.at[pl.ds(ebase + NF_DEG * CH, REM_DEG)], idx_rb)
    pltpu.sync_copy(ones_b.at[pl.ds(0, REM_DEG)], deg_sh.at[idx_rb], add=True)

    plsc.subcore_barrier()

    @pl.when(c == 0)
    def _():
        pltpu.sync_copy(deg_sh.at[pl.ds(row0, WB)], out0_r.at[pl.ds(row0, WB)])

    @pl.when(c == 1)
    def _():
        pltpu.sync_copy(deg_sh.at[pl.ds(row0, WB)], out1_r.at[pl.ds(row0, WB)])


_deg_call = pl.kernel(
    _deg_body,
    out_type=(jax.ShapeDtypeStruct((NP, 16), jnp.float32),
              jax.ShapeDtypeStruct((NP, 16), jnp.float32)),
    mesh=_mesh,
    compiler_params=pltpu.CompilerParams(use_tc_tiling_on_sc=False),
    scratch_types=(
        pltpu.VMEM((CH, 16), jnp.float32),
        pltpu.VMEM((CH,), jnp.int32),
        pltpu.VMEM((REM_DEG,), jnp.int32),
        pltpu.SemaphoreType.DMA,
        pltpu.VMEM_SHARED((NP, 16), jnp.float32),
    ),
)


def _agg_body(src_r, dst_r, glo_r, ghi_r, outlo_r, outhi_r,
              idx_s, idx_d, rows, idx_sr, idx_dr, rows_r, sem, agg_sh):
    c = lax.axis_index("c")
    s = lax.axis_index("s")
    z16 = jnp.zeros((16,), jnp.float32)

    def zb(i, carry):
        rows[i, pl.ds(0, 16)] = z16
        rows[i, pl.ds(16, 16)] = z16
        return carry

    lax.fori_loop(0, CH, zb, 0)

    row0 = s * WB

    def zcopy(k, carry):
        pltpu.sync_copy(rows, agg_sh.at[pl.ds(row0 + k * CH, CH)])
        return carry

    lax.fori_loop(0, WB // CH, zcopy, 0)
    pltpu.sync_copy(rows.at[pl.ds(0, WB - (WB // CH) * CH)],
                    agg_sh.at[pl.ds(row0 + (WB // CH) * CH, WB - (WB // CH) * CH)])
    plsc.subcore_barrier()

    ebase = s * EP_AGG

    def process(table_r):
        def step(j, carry):
            base = ebase + j * CH
            pltpu.sync_copy(src_r.at[pl.ds(base, CH)], idx_s)
            pltpu.sync_copy(dst_r.at[pl.ds(base, CH)], idx_d)
            pltpu.async_copy(table_r.at[idx_s], rows, sem).wait()
            pltpu.sync_copy(rows, agg_sh.at[idx_d], add=True)
            return carry

        lax.fori_loop(0, NF_AGG, step, 0)
        rbase = ebase + NF_AGG * CH
        pltpu.sync_copy(src_r.at[pl.ds(rbase, REM_AGG)], idx_sr)
        pltpu.sync_copy(dst_r.at[pl.ds(rbase, REM_AGG)], idx_dr)
        pltpu.async_copy(table_r.at[idx_sr], rows_r, sem).wait()
        pltpu.sync_copy(rows_r, agg_sh.at[idx_dr], add=True)

    @pl.when(c == 0)
    def _():
        process(glo_r)

    @pl.when(c == 1)
    def _():
        process(ghi_r)

    plsc.subcore_barrier()

    @pl.when(c == 0)
    def _():
        pltpu.sync_copy(agg_sh.at[pl.ds(row0, WB)], outlo_r.at[pl.ds(row0, WB)])

    @pl.when(c == 1)
    def _():
        pltpu.sync_copy(agg_sh.at[pl.ds(row0, WB)], outhi_r.at[pl.ds(row0, WB)])


_agg_call = pl.kernel(
    _agg_body,
    out_type=(jax.ShapeDtypeStruct((NP, 32), jnp.float32),
              jax.ShapeDtypeStruct((NP, 32), jnp.float32)),
    mesh=_mesh,
    compiler_params=pltpu.CompilerParams(use_tc_tiling_on_sc=False),
    scratch_types=(
        pltpu.VMEM((CH,), jnp.int32),
        pltpu.VMEM((CH,), jnp.int32),
        pltpu.VMEM((CH, 32), jnp.float32),
        pltpu.VMEM((REM_AGG,), jnp.int32),
        pltpu.VMEM((REM_AGG,), jnp.int32),
        pltpu.VMEM((REM_AGG, 32), jnp.float32),
        pltpu.SemaphoreType.DMA,
        pltpu.VMEM_SHARED((NP, 32), jnp.float32),
    ),
)


def _gru_body(x_r, wih_r, whh_r, bih_r, bhh_r, out_r):
    xb = x_r[...]
    wih = wih_r[...]
    whh = whh_r[...]
    bih = bih_r[...]
    bhh = bhh_r[...]
    h = jnp.zeros((BN, H), jnp.float32)
    for t in range(T):
        xt = lax.slice(xb, (0, t), (BN, t + 1))
        gi = xt * wih + bih
        gh = jnp.dot(h, whh, preferred_element_type=jnp.float32) + bhh
        r = jax.nn.sigmoid(gi[:, 0:H] + gh[:, 0:H])
        z = jax.nn.sigmoid(gi[:, H:2 * H] + gh[:, H:2 * H])
        n = jnp.tanh(gi[:, 2 * H:3 * H] + r * gh[:, 2 * H:3 * H])
        h = (1.0 - z) * n + z * h
    out_r[...] = h


_gru_call = pl.pallas_call(
    _gru_body,
    grid=(GRID,),
    in_specs=[
        pl.BlockSpec((BN, T), lambda i: (i, 0)),
        pl.BlockSpec((1, 3 * H), lambda i: (0, 0)),
        pl.BlockSpec((H, 3 * H), lambda i: (0, 0)),
        pl.BlockSpec((1, 3 * H), lambda i: (0, 0)),
        pl.BlockSpec((1, 3 * H), lambda i: (0, 0)),
    ],
    out_specs=pl.BlockSpec((BN, H), lambda i: (i, 0)),
    out_shape=jax.ShapeDtypeStruct((N, H), jnp.float32),
    compiler_params=pltpu.CompilerParams(
        dimension_semantics=("arbitrary",)),
)


def _dinv_of(d0, d1):
    deg = d0[:, 0:1] + d1[:, 0:1] + 1.0
    return lax.rsqrt(jnp.maximum(deg, 1e-12))


def _prep_body(h_r, d0_r, d1_r, w_r, glo_r, ghi_r):
    dinv = _dinv_of(d0_r[...], d1_r[...])
    g = dinv * jnp.dot(h_r[...], w_r[...], preferred_element_type=jnp.float32)
    glo_r[...] = g[:, 0:32]
    ghi_r[...] = g[:, 32:64]


_prep_call = pl.pallas_call(
    _prep_body,
    grid=(GRID,),
    in_specs=[
        pl.BlockSpec((BN, H), lambda i: (i, 0)),
        pl.BlockSpec((BN, 16), lambda i: (i, 0)),
        pl.BlockSpec((BN, 16), lambda i: (i, 0)),
        pl.BlockSpec((H, H), lambda i: (0, 0)),
    ],
    out_specs=[
        pl.BlockSpec((BN, 32), lambda i: (i, 0)),
        pl.BlockSpec((BN, 32), lambda i: (i, 0)),
    ],
    out_shape=(jax.ShapeDtypeStruct((N, 32), jnp.float32),
               jax.ShapeDtypeStruct((N, 32), jnp.float32)),
    compiler_params=pltpu.CompilerParams(
        dimension_semantics=("arbitrary",)),
)


def _mid_body(alo_r, ahi_r, glo_r, ghi_r, d0_r, d1_r, b1_r, w2_r,
              g2lo_r, g2hi_r):
    dinv = _dinv_of(d0_r[...], d1_r[...])
    b1 = b1_r[...]
    h1lo = jax.nn.relu(dinv * (alo_r[...] + glo_r[...]) + b1[:, 0:32])
    h1hi = jax.nn.relu(dinv * (ahi_r[...] + ghi_r[...]) + b1[:, 32:64])
    h1 = jnp.concatenate([h1lo, h1hi], axis=1)
    g2 = dinv * jnp.dot(h1, w2_r[...], preferred_element_type=jnp.float32)
    g2lo_r[...] = g2[:, 0:32]
    g2hi_r[...] = g2[:, 32:64]


_mid_call = pl.pallas_call(
    _mid_body,
    grid=(GRID,),
    in_specs=[
        pl.BlockSpec((BN, 32), lambda i: (i, 0)),
        pl.BlockSpec((BN, 32), lambda i: (i, 0)),
        pl.BlockSpec((BN, 32), lambda i: (i, 0)),
        pl.BlockSpec((BN, 32), lambda i: (i, 0)),
        pl.BlockSpec((BN, 16), lambda i: (i, 0)),
        pl.BlockSpec((BN, 16), lambda i: (i, 0)),
        pl.BlockSpec((1, H), lambda i: (0, 0)),
        pl.BlockSpec((H, H), lambda i: (0, 0)),
    ],
    out_specs=[
        pl.BlockSpec((BN, 32), lambda i: (i, 0)),
        pl.BlockSpec((BN, 32), lambda i: (i, 0)),
    ],
    out_shape=(jax.ShapeDtypeStruct((N, 32), jnp.float32),
               jax.ShapeDtypeStruct((N, 32), jnp.float32)),
    compiler_params=pltpu.CompilerParams(
        dimension_semantics=("arbitrary",)),
)


def _fin_body(alo_r, ahi_r, glo_r, ghi_r, d0_r, d1_r, b2_r, fw_r, fb_r,
              out_r):
    dinv = _dinv_of(d0_r[...], d1_r[...])
    b2 = b2_r[...]
    h2lo = jax.nn.relu(dinv * (alo_r[...] + glo_r[...]) + b2[:, 0:32])
    h2hi = jax.nn.relu(dinv * (ahi_r[...] + ghi_r[...]) + b2[:, 32:64])
    h2 = jnp.concatenate([h2lo, h2hi], axis=1)
    out_r[...] = jnp.dot(h2, fw_r[...],
                         preferred_element_type=jnp.float32) + fb_r[...]


_fin_call = pl.pallas_call(
    _fin_body,
    grid=(GRID,),
    in_specs=[
        pl.BlockSpec((BN, 32), lambda i: (i, 0)),
        pl.BlockSpec((BN, 32), lambda i: (i, 0)),
        pl.BlockSpec((BN, 32), lambda i: (i, 0)),
        pl.BlockSpec((BN, 32), lambda i: (i, 0)),
        pl.BlockSpec((BN, 16), lambda i: (i, 0)),
        pl.BlockSpec((BN, 16), lambda i: (i, 0)),
        pl.BlockSpec((1, H), lambda i: (0, 0)),
        pl.BlockSpec((H, HOR), lambda i: (0, 0)),
        pl.BlockSpec((1, HOR), lambda i: (0, 0)),
    ],
    out_specs=pl.BlockSpec((BN, HOR), lambda i: (i, 0)),
    out_shape=jax.ShapeDtypeStruct((N, HOR), jnp.float32),
    compiler_params=pltpu.CompilerParams(
        dimension_semantics=("arbitrary",)),
)


def kernel(x, edge_index, W_ih, W_hh, b_ih, b_hh, gc1_W, gc1_b, gc2_W, gc2_b,
           fc_W, fc_b):
    src = edge_index[0]
    dst = edge_index[1]
    wih_row = W_ih.reshape(1, 3 * H)
    whhT = W_hh.T
    bih2 = b_ih.reshape(1, 3 * H)
    bhh2 = b_hh.reshape(1, 3 * H)
    b1 = gc1_b.reshape(1, H)
    b2 = gc2_b.reshape(1, H)
    fwT = fc_W.T
    fb2 = fc_b.reshape(1, HOR)

    deg0, deg1 = _deg_call(dst)
    h = _gru_call(x, wih_row, whhT, bih2, bhh2)
    g1lo, g1hi = _prep_call(h, deg0, deg1, gc1_W)
    a1lo, a1hi = _agg_call(src, dst, g1lo, g1hi)
    g2lo, g2hi = _mid_call(a1lo, a1hi, g1lo, g1hi, deg0, deg1, b1, gc2_W)
    a2lo, a2hi = _agg_call(src, dst, g2lo, g2hi)
    out = _fin_call(a2lo, a2hi, g2lo, g2hi, deg0, deg1, b2, fwT, fb2)
    return out


# speedup vs baseline: 15.3410x; 1.3630x over previous
import functools

import jax
import jax.numpy as jnp
from jax import lax
from jax.experimental import pallas as pl
from jax.experimental.pallas import tpu as pltpu
from jax.experimental.pallas import tpu_sc as plsc

N = 50000
T = 12
H = 64
HOR = 12
E = 800000

NTILE = 16
NCORE = 2
WB = 3128
NP = NTILE * WB
CH = 128
TRASH = N

NCHUNK = 6400
EPAD = NCHUNK * CH - E
GA = 2
GD = 8

RPT_AGG = NCHUNK // NTILE
NG_AGG = RPT_AGG // GA

RPT_DEG = NCHUNK // (NTILE * NCORE)
NG_DEG = RPT_DEG // GD

BN = 1000
GRID = N // BN

_mesh = plsc.VectorSubcoreMesh(core_axis_name="c", subcore_axis_name="s")


def _deg_body(eidx_r, out0_r, out1_r, ones_b, ebuf, sem_i, sem_s, deg_sh):
    c = lax.axis_index("c")
    s = lax.axis_index("s")
    z16 = jnp.zeros((16,), jnp.float32)
    o16 = jnp.ones((16,), jnp.float32)

    def fill(i, carry):
        ones_b[i, pl.ds(0, 16)] = z16
        return carry

    lax.fori_loop(0, CH, fill, 0)

    row0 = s * WB

    def zcopy(k, carry):
        pltpu.sync_copy(ones_b, deg_sh.at[pl.ds(row0 + k * CH, CH)])
        return carry

    lax.fori_loop(0, WB // CH, zcopy, 0)
    pltpu.sync_copy(ones_b.at[pl.ds(0, WB - (WB // CH) * CH)],
                    deg_sh.at[pl.ds(row0 + (WB // CH) * CH, WB - (WB // CH) * CH)])

    def refill(i, carry):
        ones_b[i, pl.ds(0, 16)] = o16
        return carry

    lax.fori_loop(0, CH, refill, 0)
    plsc.subcore_barrier()

    r0 = (c * NTILE + s) * RPT_DEG
    pltpu.async_copy(eidx_r.at[pl.ds(r0, GD)], ebuf.at[pl.ds(0, GD)],
                     sem_i.at[0])

    def group(g, carry):
        p = lax.rem(g, 2)
        q = 1 - p
        pltpu.make_async_copy(eidx_r.at[pl.ds(r0, GD)],
                              ebuf.at[pl.ds(p * GD, GD)], sem_i.at[p]).wait()
        for b in range(GD):
            pltpu.async_copy(ones_b, deg_sh.at[ebuf.at[p * GD + b, 1]],
                             sem_s.at[p], add=True)
        @pl.when(g >= 1)
        def _():
            for b in range(GD):
                pltpu.make_async_copy(
                    ones_b, deg_sh.at[pl.ds(0, CH)], sem_s.at[q]).wait()
        @pl.when(g + 1 < NG_DEG)
        def _():
            pltpu.async_copy(eidx_r.at[pl.ds(r0 + (g + 1) * GD, GD)],
                             ebuf.at[pl.ds(q * GD, GD)], sem_i.at[q])
        return carry

    lax.fori_loop(0, NG_DEG, group, 0)
    for b in range(GD):
        pltpu.make_async_copy(ones_b, deg_sh.at[pl.ds(0, CH)],
                              sem_s.at[(NG_DEG - 1) % 2]).wait()

    plsc.subcore_barrier()

    @pl.when(c == 0)
    def _():
        pltpu.sync_copy(deg_sh.at[pl.ds(row0, WB)], out0_r.at[pl.ds(row0, WB)])

    @pl.when(c == 1)
    def _():
        pltpu.sync_copy(deg_sh.at[pl.ds(row0, WB)], out1_r.at[pl.ds(row0, WB)])


_deg_call = pl.kernel(
    _deg_body,
    out_type=(jax.ShapeDtypeStruct((NP, 16), jnp.float32),
              jax.ShapeDtypeStruct((NP, 16), jnp.float32)),
    mesh=_mesh,
    compiler_params=pltpu.CompilerParams(use_tc_tiling_on_sc=False),
    scratch_types=(
        pltpu.VMEM((CH, 16), jnp.float32),
        pltpu.VMEM((2 * GD, 2, CH), jnp.int32),
        pltpu.SemaphoreType.DMA((2,)),
        pltpu.SemaphoreType.DMA((2,)),
        pltpu.VMEM_SHARED((NP, 16), jnp.float32),
    ),
)


def _agg_body(eidx_r, glo_r, ghi_r, outlo_r, outhi_r,
              ebuf, rows, sem_i, sem_g, sem_s, agg_sh):
    c = lax.axis_index("c")
    s = lax.axis_index("s")
    z16 = jnp.zeros((16,), jnp.float32)

    def zb(i, carry):
        rows[i, pl.ds(0, 16)] = z16
        rows[i, pl.ds(16, 16)] = z16
        return carry

    lax.fori_loop(0, CH, zb, 0)

    row0 = s * WB

    def zcopy(k, carry):
        pltpu.sync_copy(rows.at[pl.ds(0, CH)],
                        agg_sh.at[pl.ds(row0 + k * CH, CH)])
        return carry

    lax.fori_loop(0, WB // CH, zcopy, 0)
    pltpu.sync_copy(rows.at[pl.ds(0, WB - (WB // CH) * CH)],
                    agg_sh.at[pl.ds(row0 + (WB // CH) * CH, WB - (WB // CH) * CH)])
    plsc.subcore_barrier()

    r0 = s * RPT_AGG

    def process(table_r):
        pltpu.async_copy(eidx_r.at[pl.ds(r0, GA)], ebuf.at[pl.ds(0, GA)],
                         sem_i.at[0])

        def group(g, carry):
            p = lax.rem(g, 2)
            q = 1 - p
            pltpu.make_async_copy(eidx_r.at[pl.ds(r0, GA)],
                                  ebuf.at[pl.ds(p * GA, GA)], sem_i.at[p]).wait()
            gds = []
            for b in range(GA):
                gds.append(pltpu.async_copy(
                    table_r.at[ebuf.at[p * GA + b, 0]],
                    rows.at[pl.ds((p * GA + b) * CH, CH)], sem_g.at[p]))
            @pl.when(g >= 1)
            def _():
                for b in range(GA):
                    pltpu.make_async_copy(
                        rows.at[pl.ds((q * GA + b) * CH, CH)],
                        agg_sh.at[pl.ds(0, CH)], sem_s.at[q]).wait()
            @pl.when(g + 1 < NG_AGG)
            def _():
                pltpu.async_copy(eidx_r.at[pl.ds(r0 + (g + 1) * GA, GA)],
                                 ebuf.at[pl.ds(q * GA, GA)], sem_i.at[q])
            for d in gds:
                d.wait()
            for b in range(GA):
                pltpu.async_copy(rows.at[pl.ds((p * GA + b) * CH, CH)],
                                 agg_sh.at[ebuf.at[p * GA + b, 1]],
                                 sem_s.at[p], add=True)
            return carry

        lax.fori_loop(0, NG_AGG, group, 0)
        for b in range(GA):
            pltpu.make_async_copy(
                rows.at[pl.ds(b * CH, CH)],
                agg_sh.at[pl.ds(0, CH)], sem_s.at[(NG_AGG - 1) % 2]).wait()

    @pl.when(c == 0)
    def _():
        process(glo_r)

    @pl.when(c == 1)
    def _():
        process(ghi_r)

    plsc.subcore_barrier()

    @pl.when(c == 0)
    def _():
        pltpu.sync_copy(agg_sh.at[pl.ds(row0, WB)], outlo_r.at[pl.ds(row0, WB)])

    @pl.when(c == 1)
    def _():
        pltpu.sync_copy(agg_sh.at[pl.ds(row0, WB)], outhi_r.at[pl.ds(row0, WB)])


_agg_call = pl.kernel(
    _agg_body,
    out_type=(jax.ShapeDtypeStruct((NP, 32), jnp.float32),
              jax.ShapeDtypeStruct((NP, 32), jnp.float32)),
    mesh=_mesh,
    compiler_params=pltpu.CompilerParams(use_tc_tiling_on_sc=False),
    scratch_types=(
        pltpu.VMEM((2 * GA, 2, CH), jnp.int32),
        pltpu.VMEM((2 * GA * CH, 32), jnp.float32),
        pltpu.SemaphoreType.DMA((2,)),
        pltpu.SemaphoreType.DMA((2,)),
        pltpu.SemaphoreType.DMA((2,)),
        pltpu.VMEM_SHARED((NP, 32), jnp.float32),
    ),
)


def _gru_body(x_r, wih_r, whh_r, bih_r, bhh_r, out_r):
    xb = x_r[...]
    wih = wih_r[...]
    whh = whh_r[...]
    bih = bih_r[...]
    bhh = bhh_r[...]
    h = jnp.zeros((BN, H), jnp.float32)
    for t in range(T):
        xt = lax.slice(xb, (0, t), (BN, t + 1))
        gi = xt * wih + bih
        gh = jnp.dot(h, whh, preferred_element_type=jnp.float32) + bhh
        r = jax.nn.sigmoid(gi[:, 0:H] + gh[:, 0:H])
        z = jax.nn.sigmoid(gi[:, H:2 * H] + gh[:, H:2 * H])
        n = jnp.tanh(gi[:, 2 * H:3 * H] + r * gh[:, 2 * H:3 * H])
        h = (1.0 - z) * n + z * h
    out_r[...] = h


_gru_call = pl.pallas_call(
    _gru_body,
    grid=(GRID,),
    in_specs=[
        pl.BlockSpec((BN, T), lambda i: (i, 0)),
        pl.BlockSpec((1, 3 * H), lambda i: (0, 0)),
        pl.BlockSpec((H, 3 * H), lambda i: (0, 0)),
        pl.BlockSpec((1, 3 * H), lambda i: (0, 0)),
        pl.BlockSpec((1, 3 * H), lambda i: (0, 0)),
    ],
    out_specs=pl.BlockSpec((BN, H), lambda i: (i, 0)),
    out_shape=jax.ShapeDtypeStruct((N, H), jnp.float32),
    compiler_params=pltpu.CompilerParams(
        dimension_semantics=("arbitrary",)),
)


def _dinv_of(d0, d1):
    deg = d0[:, 0:1] + d1[:, 0:1] + 1.0
    return lax.rsqrt(jnp.maximum(deg, 1e-12))


def _prep_body(h_r, d0_r, d1_r, w_r, glo_r, ghi_r):
    dinv = _dinv_of(d0_r[...], d1_r[...])
    g = dinv * jnp.dot(h_r[...], w_r[...], preferred_element_type=jnp.float32)
    glo_r[...] = g[:, 0:32]
    ghi_r[...] = g[:, 32:64]


_prep_call = pl.pallas_call(
    _prep_body,
    grid=(GRID,),
    in_specs=[
        pl.BlockSpec((BN, H), lambda i: (i, 0)),
        pl.BlockSpec((BN, 16), lambda i: (i, 0)),
        pl.BlockSpec((BN, 16), lambda i: (i, 0)),
        pl.BlockSpec((H, H), lambda i: (0, 0)),
    ],
    out_specs=[
        pl.BlockSpec((BN, 32), lambda i: (i, 0)),
        pl.BlockSpec((BN, 32), lambda i: (i, 0)),
    ],
    out_shape=(jax.ShapeDtypeStruct((N, 32), jnp.float32),
               jax.ShapeDtypeStruct((N, 32), jnp.float32)),
    compiler_params=pltpu.CompilerParams(
        dimension_semantics=("arbitrary",)),
)


def _mid_body(alo_r, ahi_r, glo_r, ghi_r, d0_r, d1_r, b1_r, w2_r,
              g2lo_r, g2hi_r):
    dinv = _dinv_of(d0_r[...], d1_r[...])
    b1 = b1_r[...]
    h1lo = jax.nn.relu(dinv * (alo_r[...] + glo_r[...]) + b1[:, 0:32])
    h1hi = jax.nn.relu(dinv * (ahi_r[...] + ghi_r[...]) + b1[:, 32:64])
    h1 = jnp.concatenate([h1lo, h1hi], axis=1)
    g2 = dinv * jnp.dot(h1, w2_r[...], preferred_element_type=jnp.float32)
    g2lo_r[...] = g2[:, 0:32]
    g2hi_r[...] = g2[:, 32:64]


_mid_call = pl.pallas_call(
    _mid_body,
    grid=(GRID,),
    in_specs=[
        pl.BlockSpec((BN, 32), lambda i: (i, 0)),
        pl.BlockSpec((BN, 32), lambda i: (i, 0)),
        pl.BlockSpec((BN, 32), lambda i: (i, 0)),
        pl.BlockSpec((BN, 32), lambda i: (i, 0)),
        pl.BlockSpec((BN, 16), lambda i: (i, 0)),
        pl.BlockSpec((BN, 16), lambda i: (i, 0)),
        pl.BlockSpec((1, H), lambda i: (0, 0)),
        pl.BlockSpec((H, H), lambda i: (0, 0)),
    ],
    out_specs=[
        pl.BlockSpec((BN, 32), lambda i: (i, 0)),
        pl.BlockSpec((BN, 32), lambda i: (i, 0)),
    ],
    out_shape=(jax.ShapeDtypeStruct((N, 32), jnp.float32),
               jax.ShapeDtypeStruct((N, 32), jnp.float32)),
    compiler_params=pltpu.CompilerParams(
        dimension_semantics=("arbitrary",)),
)


def _fin_body(alo_r, ahi_r, glo_r, ghi_r, d0_r, d1_r, b2_r, fw_r, fb_r,
              out_r):
    dinv = _dinv_of(d0_r[...], d1_r[...])
    b2 = b2_r[...]
    h2lo = jax.nn.relu(dinv * (alo_r[...] + glo_r[...]) + b2[:, 0:32])
    h2hi = jax.nn.relu(dinv * (ahi_r[...] + ghi_r[...]) + b2[:, 32:64])
    h2 = jnp.concatenate([h2lo, h2hi], axis=1)
    out_r[...] = jnp.dot(h2, fw_r[...],
                         preferred_element_type=jnp.float32) + fb_r[...]


_fin_call = pl.pallas_call(
    _fin_body,
    grid=(GRID,),
    in_specs=[
        pl.BlockSpec((BN, 32), lambda i: (i, 0)),
        pl.BlockSpec((BN, 32), lambda i: (i, 0)),
        pl.BlockSpec((BN, 32), lambda i: (i, 0)),
        pl.BlockSpec((BN, 32), lambda i: (i, 0)),
        pl.BlockSpec((BN, 16), lambda i: (i, 0)),
        pl.BlockSpec((BN, 16), lambda i: (i, 0)),
        pl.BlockSpec((1, H), lambda i: (0, 0)),
        pl.BlockSpec((H, HOR), lambda i: (0, 0)),
        pl.BlockSpec((1, HOR), lambda i: (0, 0)),
    ],
    out_specs=pl.BlockSpec((BN, HOR), lambda i: (i, 0)),
    out_shape=jax.ShapeDtypeStruct((N, HOR), jnp.float32),
    compiler_params=pltpu.CompilerParams(
        dimension_semantics=("arbitrary",)),
)


def kernel(x, edge_index, W_ih, W_hh, b_ih, b_hh, gc1_W, gc1_b, gc2_W, gc2_b,
           fc_W, fc_b):
    src = jnp.concatenate([edge_index[0],
                           jnp.zeros((EPAD,), jnp.int32)])
    dst = jnp.concatenate([edge_index[1],
                           jnp.full((EPAD,), TRASH, jnp.int32)])
    eidx = jnp.stack([src.reshape(NCHUNK, CH), dst.reshape(NCHUNK, CH)],
                     axis=1)
    wih_row = W_ih.reshape(1, 3 * H)
    whhT = W_hh.T
    bih2 = b_ih.reshape(1, 3 * H)
    bhh2 = b_hh.reshape(1, 3 * H)
    b1 = gc1_b.reshape(1, H)
    b2 = gc2_b.reshape(1, H)
    fwT = fc_W.T
    fb2 = fc_b.reshape(1, HOR)

    deg0, deg1 = _deg_call(eidx)
    h = _gru_call(x, wih_row, whhT, bih2, bhh2)
    g1lo, g1hi = _prep_call(h, deg0, deg1, gc1_W)
    a1lo, a1hi = _agg_call(eidx, g1lo, g1hi)
    g2lo, g2hi = _mid_call(a1lo, a1hi, g1lo, g1hi, deg0, deg1, b1, gc2_W)
    a2lo, a2hi = _agg_call(eidx, g2lo, g2hi)
    out = _fin_call(a2lo, a2hi, g2lo, g2hi, deg0, deg1, b2, fwT, fb2)
    return out
